# baseline (speedup 1.0000x reference)
import functools

import jax
import jax.numpy as jnp
from jax import lax
from jax.experimental import pallas as pl
from jax.experimental.pallas import tpu as pltpu
from jax.experimental.pallas import tpu_sc as plsc

EMBED = 2048
NEXP = 16
S_BLK = 1024
CAP = 512


def _scores_body(x_ref, w_ref, o_ref):
    x = x_ref[0]
    w = w_ref[...]
    logits = jax.lax.dot_general(w, x, (((1,), (1,)), ((), ())))
    m = jnp.max(logits, axis=0, keepdims=True)
    e = jnp.exp(logits - m)
    s = jnp.sum(e, axis=0, keepdims=True)
    o_ref[0] = e / s


def _scores(hidden_states, weight):
    B, S, d = hidden_states.shape
    grid = (B, S // S_BLK)
    return pl.pallas_call(
        _scores_body,
        grid=grid,
        in_specs=[
            pl.BlockSpec((1, S_BLK, d), lambda b, sb: (b, sb, 0)),
            pl.BlockSpec((NEXP, d), lambda b, sb: (0, 0)),
        ],
        out_specs=pl.BlockSpec((1, NEXP, S_BLK), lambda b, sb: (b, 0, sb)),
        out_shape=jax.ShapeDtypeStruct((B, NEXP, S), jnp.float32),
    )(hidden_states, weight)


def _find_thresh(histref, nv, target):
    def body(j, carry):
        run, best = carry
        i = nv - 1 - j
        h = histref[pl.ds(i * 16, 16)]
        sfx = lax.rev(plsc.cumsum(lax.rev(h, (0,))), (0,)) + run
        idxs = lax.iota(jnp.int32, 16) + i * 16
        cand = jnp.max(jnp.where(sfx >= target, idxs, -1), axis=0)
        return run + jnp.sum(h, axis=0), jnp.maximum(best, cand)
    _, best = lax.fori_loop(0, nv, body, (jnp.int32(0), jnp.int32(-1)))
    return best


def _count_above(histref, nv, b):
    def body(i, acc):
        h = histref[pl.ds(i * 16, 16)]
        idxs = lax.iota(jnp.int32, 16) + i * 16
        return acc + jnp.sum(jnp.where(idxs > b, h, 0), axis=0)
    return lax.fori_loop(0, nv, body, jnp.int32(0))


def _sc_topk(scores_flat):
    R, S = scores_flat.shape
    NV = S // 16
    mesh = plsc.VectorSubcoreMesh(core_axis_name="c", subcore_axis_name="s")

    @functools.partial(
        pl.kernel, mesh=mesh,
        out_type=(jax.ShapeDtypeStruct((R, CAP), jnp.int32),
                  jax.ShapeDtypeStruct((R, CAP), jnp.float32)),
        scratch_types=[
            pltpu.VMEM((S,), jnp.float32),
            pltpu.VMEM((1024,), jnp.int32),
            pltpu.VMEM((S,), jnp.int32),
            pltpu.VMEM((S,), jnp.int32),
            pltpu.VMEM((CAP,), jnp.int32),
            pltpu.VMEM((CAP,), jnp.int32),
            pltpu.VMEM((CAP,), jnp.int32),
            pltpu.VMEM((CAP,), jnp.int32),
            pltpu.VMEM((CAP,), jnp.float32),
            pltpu.VMEM((32,), jnp.int32),
            pltpu.VMEM((32,), jnp.int32),
        ],
        compiler_params=pltpu.CompilerParams(needs_layout_passes=False),
    )
    def topk_kernel(x_hbm, idx_hbm, w_hbm, key_f, hist, cand_k, cand_i,
                    selA_k, selA_i, selB_k, selB_i, sel_kf, h32, off32):
        wid = lax.axis_index("s") * 2 + lax.axis_index("c")
        iota = lax.iota(jnp.int32, 16)
        ones = jnp.ones((16,), jnp.int32)
        zeros = jnp.zeros((16,), jnp.int32)

        def zero_hist(_i, _):
            hist[pl.ds(_i * 16, 16)] = zeros
            return 0

        def do_row(r, _):
            row = wid * 2 + r
            pltpu.sync_copy(x_hbm.at[row], key_f)

            lax.fori_loop(0, 64, zero_hist, 0)

            def hist1(i, _):
                v = plsc.bitcast(key_f[pl.ds(i * 16, 16)], jnp.int32)
                plsc.addupdate_scatter(hist, [v >> 20], ones)
                return 0
            lax.fori_loop(0, NV, hist1, 0)

            b1 = _find_thresh(hist, 64, jnp.int32(CAP))

            lax.fori_loop(0, 64, zero_hist, 0)

            def compact1(i, carry):
                run_gt, run_eq = carry
                v = plsc.bitcast(key_f[pl.ds(i * 16, 16)], jnp.int32)
                d = v >> 20
                gidx = iota + i * 16
                mgt = d > b1
                meq = d == b1
                cg = plsc.cumsum(jnp.where(mgt, 1, 0))
                ce = plsc.cumsum(jnp.where(meq, 1, 0))
                pg = run_gt + cg - 1
                pe = run_eq + ce - 1
                plsc.store_scatter(selA_k, [pg], v, mask=mgt)
                plsc.store_scatter(selA_i, [pg], gidx, mask=mgt)
                plsc.store_scatter(cand_k, [pe], v, mask=meq)
                plsc.store_scatter(cand_i, [pe], gidx, mask=meq)
                plsc.addupdate_scatter(hist, [(v >> 10) & 0x3FF], ones,
                                       mask=meq)
                return (run_gt + jnp.sum(jnp.where(mgt, 1, 0), axis=0),
                        run_eq + jnp.sum(jnp.where(meq, 1, 0), axis=0))
            run_gt, n1 = lax.fori_loop(0, NV, compact1,
                                       (jnp.int32(0), jnp.int32(0)))

            b2 = _find_thresh(hist, 64, jnp.int32(CAP) - run_gt)

            lax.fori_loop(0, 64, zero_hist, 0)
            nv1 = (n1 + 15) >> 4

            def compact2(i, carry):
                run_gt, run_eq = carry
                v = cand_k[pl.ds(i * 16, 16)]
                gi = cand_i[pl.ds(i * 16, 16)]
                ok = (iota + i * 16) < n1
                d = (v >> 10) & 0x3FF
                mgt = ok & (d > b2)
                meq = ok & (d == b2)
                cg = plsc.cumsum(jnp.where(mgt, 1, 0))
                ce = plsc.cumsum(jnp.where(meq, 1, 0))
                pg = run_gt + cg - 1
                pe = run_eq + ce - 1
                plsc.store_scatter(selA_k, [pg], v, mask=mgt)
                plsc.store_scatter(selA_i, [pg], gi, mask=mgt)
                plsc.store_scatter(cand_k, [pe], v, mask=meq)
                plsc.store_scatter(cand_i, [pe], gi, mask=meq)
                plsc.addupdate_scatter(hist, [v & 0x3FF], ones, mask=meq)
                return (run_gt + jnp.sum(jnp.where(mgt, 1, 0), axis=0),
                        run_eq + jnp.sum(jnp.where(meq, 1, 0), axis=0))
            run_gt2, n2 = lax.fori_loop(0, nv1, compact2,
                                        (run_gt, jnp.int32(0)))

            r2 = jnp.int32(CAP) - run_gt2
            b3 = _find_thresh(hist, 64, r2)
            cnt3 = _count_above(hist, 64, b3)
            eq_base = jnp.int32(CAP) - (r2 - cnt3)

            nv2 = (n2 + 15) >> 4

            def compact3(i, carry):
                run_gt, run_eq = carry
                v = cand_k[pl.ds(i * 16, 16)]
                gi = cand_i[pl.ds(i * 16, 16)]
                ok = (iota + i * 16) < n2
                d = v & 0x3FF
                mgt = ok & (d > b3)
                meq = ok & (d == b3)
                cg = plsc.cumsum(jnp.where(mgt, 1, 0))
                ce = plsc.cumsum(jnp.where(meq, 1, 0))
                pg = run_gt + cg - 1
                pe = eq_base + run_eq + ce - 1
                mst = meq & (pe < CAP)
                plsc.store_scatter(selA_k, [pg], v, mask=mgt)
                plsc.store_scatter(selA_i, [pg], gi, mask=mgt)
                plsc.store_scatter(selA_k, [pe], v, mask=mst)
                plsc.store_scatter(selA_i, [pe], gi, mask=mst)
                return (run_gt + jnp.sum(jnp.where(mgt, 1, 0), axis=0),
                        run_eq + jnp.sum(jnp.where(meq, 1, 0), axis=0))
            lax.fori_loop(0, nv2, compact3, (run_gt2, jnp.int32(0)))

            bufs = [(selA_k, selA_i), (selB_k, selB_i)]
            for p in range(6):
                src_k, src_i = bufs[p % 2]
                dst_k, dst_i = bufs[(p + 1) % 2]
                h32[pl.ds(0, 16)] = zeros
                h32[pl.ds(16, 16)] = zeros

                def histp(i, _, src_k=src_k, p=p):
                    k = src_k[pl.ds(i * 16, 16)]
                    d = 31 - ((k >> (5 * p)) & 31)
                    plsc.addupdate_scatter(h32, [d], ones)
                    return 0
                lax.fori_loop(0, CAP // 16, histp, 0)

                h0 = h32[pl.ds(0, 16)]
                h1 = h32[pl.ds(16, 16)]
                c0 = plsc.cumsum(h0)
                c1 = plsc.cumsum(h1) + jnp.sum(h0, axis=0)
                off32[pl.ds(0, 16)] = c0 - h0
                off32[pl.ds(16, 16)] = c1 - h1

                def permp(i, _, src_k=src_k, src_i=src_i,
                          dst_k=dst_k, dst_i=dst_i, p=p):
                    k = src_k[pl.ds(i * 16, 16)]
                    gi = src_i[pl.ds(i * 16, 16)]
                    d = 31 - ((k >> (5 * p)) & 31)
                    occ, lastm = plsc.scan_count(d)
                    base = plsc.load_gather(off32, [d])
                    pos = base + occ - 1
                    if p == 5:
                        plsc.store_scatter(sel_kf, [pos],
                                           plsc.bitcast(k, jnp.float32))
                    else:
                        plsc.store_scatter(dst_k, [pos], k)
                    plsc.store_scatter(dst_i, [pos], gi)
                    plsc.addupdate_scatter(off32, [d], occ, mask=lastm)
                    return 0
                lax.fori_loop(0, CAP // 16, permp, 0)

            pltpu.sync_copy(sel_kf, w_hbm.at[row])
            pltpu.sync_copy(selA_i, idx_hbm.at[row])
            return 0

        lax.fori_loop(0, 2, do_row, 0)

    return topk_kernel(scores_flat)


def kernel(hidden_states, weight):
    B, S, d = hidden_states.shape
    scores = _scores(hidden_states, weight)
    idx_flat, wgt_flat = _sc_topk(scores.reshape(B * NEXP, S))
    return (idx_flat.reshape(B, NEXP, CAP),
            wgt_flat.reshape(B, NEXP, CAP))


# speedup vs baseline: 1.1829x; 1.1829x over previous
import functools

import jax
import jax.numpy as jnp
from jax import lax
from jax.experimental import pallas as pl
from jax.experimental.pallas import tpu as pltpu
from jax.experimental.pallas import tpu_sc as plsc

EMBED = 2048
NEXP = 16
S_BLK = 1024
CAP = 512


def _scores_body(x_ref, w_ref, o_ref):
    x = x_ref[0]
    w = w_ref[...]
    logits = jax.lax.dot_general(w, x, (((1,), (1,)), ((), ())))
    m = jnp.max(logits, axis=0, keepdims=True)
    e = jnp.exp(logits - m)
    s = jnp.sum(e, axis=0, keepdims=True)
    o_ref[0] = e / s


def _scores(hidden_states, weight):
    B, S, d = hidden_states.shape
    grid = (B, S // S_BLK)
    return pl.pallas_call(
        _scores_body,
        grid=grid,
        in_specs=[
            pl.BlockSpec((1, S_BLK, d), lambda b, sb: (b, sb, 0)),
            pl.BlockSpec((NEXP, d), lambda b, sb: (0, 0)),
        ],
        out_specs=pl.BlockSpec((1, NEXP, S_BLK), lambda b, sb: (b, 0, sb)),
        out_shape=jax.ShapeDtypeStruct((B, NEXP, S), jnp.float32),
    )(hidden_states, weight)


def _popcnt(mask):
    return plsc.all_reduce_population_count(mask)


def _lane_gather(x, idx):
    return lax.gather(
        x, idx[:, None],
        dimension_numbers=lax.GatherDimensionNumbers(
            offset_dims=(), collapsed_slice_dims=(0,), start_index_map=(0,)),
        slice_sizes=(1,),
        mode=lax.GatherScatterMode.PROMISE_IN_BOUNDS)


def _find_thresh(histref, target_v):
    iota = lax.iota(jnp.int32, 16)
    zeros = jnp.zeros((16,), jnp.int32)
    fifteen = jnp.full((16,), 15, jnp.int32)

    def body(j, carry):
        runv, bestv, cntv = carry
        i = 63 - j
        h = histref[pl.ds(i * 16, 16)]
        histref[pl.ds(i * 16, 16)] = zeros
        d = plsc.cumsum(lax.rev(h, (0,)))
        sfx = lax.rev(d, (0,)) + runv
        mask = sfx >= target_v
        idxs = iota + i * 16
        bestv = jnp.maximum(bestv, jnp.where(mask, idxs, -1))
        cntv = jnp.maximum(cntv, jnp.where(mask, -1, sfx))
        runv = runv + _lane_gather(d, fifteen)
        return runv, bestv, cntv

    init = (zeros, jnp.full((16,), -1, jnp.int32), zeros)
    _, bestv, cntv = plsc.parallel_loop(0, 64, carry=init)(body)
    best = jnp.full((16,), jnp.max(bestv, axis=0), jnp.int32)
    cnt = jnp.full((16,), jnp.maximum(jnp.max(cntv, axis=0), 0), jnp.int32)
    return best, cnt


def _sc_topk(scores_flat):
    R, S = scores_flat.shape
    NV = S // 16
    mesh = plsc.VectorSubcoreMesh(core_axis_name="c", subcore_axis_name="s")

    @functools.partial(
        pl.kernel, mesh=mesh,
        out_type=(jax.ShapeDtypeStruct((R, CAP), jnp.int32),
                  jax.ShapeDtypeStruct((R, CAP), jnp.float32)),
        scratch_types=[
            pltpu.VMEM((S,), jnp.float32),
            pltpu.VMEM((1024,), jnp.int32),
            pltpu.VMEM((S,), jnp.int32),
            pltpu.VMEM((S,), jnp.int32),
            pltpu.VMEM((CAP,), jnp.int32),
            pltpu.VMEM((CAP,), jnp.int32),
            pltpu.VMEM((CAP,), jnp.int32),
            pltpu.VMEM((CAP,), jnp.int32),
            pltpu.VMEM((CAP,), jnp.float32),
            pltpu.VMEM((32,), jnp.int32),
            pltpu.VMEM((32,), jnp.int32),
        ],
        compiler_params=pltpu.CompilerParams(needs_layout_passes=False),
    )
    def topk_kernel(x_hbm, idx_hbm, w_hbm, key_f, hist, cand_k, cand_i,
                    selA_k, selA_i, selB_k, selB_i, sel_kf, h32, off32):
        wid = lax.axis_index("s") * 2 + lax.axis_index("c")
        iota = lax.iota(jnp.int32, 16)
        ones = jnp.ones((16,), jnp.int32)
        zeros = jnp.zeros((16,), jnp.int32)
        capv = jnp.full((16,), CAP, jnp.int32)

        def zero_hist(i):
            hist[pl.ds(i * 16, 16)] = zeros
        plsc.parallel_loop(0, 64, unroll=4)(zero_hist)

        def do_row(r, _):
            row = wid * 2 + r
            pltpu.sync_copy(x_hbm.at[row], key_f)

            def hist1(i):
                v = plsc.bitcast(key_f[pl.ds(i * 16, 16)], jnp.int32)
                plsc.addupdate_scatter(hist, [v >> 20], ones)
            plsc.parallel_loop(0, NV, unroll=4)(hist1)

            b1, _ = _find_thresh(hist, capv)

            def compact1(i, carry):
                rgv, rqv = carry
                v = plsc.bitcast(key_f[pl.ds(i * 16, 16)], jnp.int32)
                d = v >> 20
                gidx = iota + i * 16
                mgt = d > b1
                meq = d == b1
                cg = plsc.cumsum(jnp.where(mgt, 1, 0))
                ce = plsc.cumsum(jnp.where(meq, 1, 0))
                pg = rgv + cg - 1
                pe = rqv + ce - 1
                plsc.store_scatter(selA_k, [pg], v, mask=mgt)
                plsc.store_scatter(selA_i, [pg], gidx, mask=mgt)
                plsc.store_scatter(cand_k, [pe], v, mask=meq)
                plsc.store_scatter(cand_i, [pe], gidx, mask=meq)
                plsc.addupdate_scatter(hist, [(v >> 10) & 0x3FF], ones,
                                       mask=meq)
                return rgv + _popcnt(mgt), rqv + _popcnt(meq)
            rgv, n1v = plsc.parallel_loop(
                0, NV, unroll=2, carry=(zeros, zeros))(compact1)

            b2, _ = _find_thresh(hist, capv - rgv)

            n1 = jnp.max(n1v, axis=0)
            nv1 = (n1 + 15) >> 4

            def compact2(i, carry):
                rgv, rqv = carry
                v = cand_k[pl.ds(i * 16, 16)]
                gi = cand_i[pl.ds(i * 16, 16)]
                ok = (iota + i * 16) < n1v
                d = (v >> 10) & 0x3FF
                mgt = ok & (d > b2)
                meq = ok & (d == b2)
                cg = plsc.cumsum(jnp.where(mgt, 1, 0))
                ce = plsc.cumsum(jnp.where(meq, 1, 0))
                pg = rgv + cg - 1
                pe = rqv + ce - 1
                plsc.store_scatter(selA_k, [pg], v, mask=mgt)
                plsc.store_scatter(selA_i, [pg], gi, mask=mgt)
                plsc.store_scatter(cand_k, [pe], v, mask=meq)
                plsc.store_scatter(cand_i, [pe], gi, mask=meq)
                plsc.addupdate_scatter(hist, [v & 0x3FF], ones, mask=meq)
                return rgv + _popcnt(mgt), rqv + _popcnt(meq)
            rgv2, n2v = lax.fori_loop(0, nv1, compact2, (rgv, zeros))

            r2v = capv - rgv2
            b3, cnt3 = _find_thresh(hist, r2v)
            eq_base = capv - (r2v - cnt3)

            n2 = jnp.max(n2v, axis=0)
            nv2 = (n2 + 15) >> 4

            def compact3(i, carry):
                rgv, rqv = carry
                v = cand_k[pl.ds(i * 16, 16)]
                gi = cand_i[pl.ds(i * 16, 16)]
                ok = (iota + i * 16) < n2v
                d = v & 0x3FF
                mgt = ok & (d > b3)
                meq = ok & (d == b3)
                cg = plsc.cumsum(jnp.where(mgt, 1, 0))
                ce = plsc.cumsum(jnp.where(meq, 1, 0))
                pg = rgv + cg - 1
                pe = eq_base + rqv + ce - 1
                mst = meq & (pe < capv)
                plsc.store_scatter(selA_k, [pg], v, mask=mgt)
                plsc.store_scatter(selA_i, [pg], gi, mask=mgt)
                plsc.store_scatter(selA_k, [pe], v, mask=mst)
                plsc.store_scatter(selA_i, [pe], gi, mask=mst)
                return rgv + _popcnt(mgt), rqv + _popcnt(meq)
            lax.fori_loop(0, nv2, compact3, (rgv2, zeros))

            bufs = [(selA_k, selA_i), (selB_k, selB_i)]
            for p in range(6):
                src_k, src_i = bufs[p % 2]
                dst_k, dst_i = bufs[(p + 1) % 2]
                h32[pl.ds(0, 16)] = zeros
                h32[pl.ds(16, 16)] = zeros

                def histp(i, src_k=src_k, p=p):
                    k = src_k[pl.ds(i * 16, 16)]
                    d = 31 - ((k >> (5 * p)) & 31)
                    plsc.addupdate_scatter(h32, [d], ones)
                plsc.parallel_loop(0, CAP // 16, unroll=4)(histp)

                h0 = h32[pl.ds(0, 16)]
                h1 = h32[pl.ds(16, 16)]
                c0 = plsc.cumsum(h0)
                c1 = plsc.cumsum(h1) + jnp.sum(h0, axis=0)
                off32[pl.ds(0, 16)] = c0 - h0
                off32[pl.ds(16, 16)] = c1 - h1

                def permp(i, _, src_k=src_k, src_i=src_i,
                          dst_k=dst_k, dst_i=dst_i, p=p):
                    k = src_k[pl.ds(i * 16, 16)]
                    gi = src_i[pl.ds(i * 16, 16)]
                    d = 31 - ((k >> (5 * p)) & 31)
                    occ, lastm = plsc.scan_count(d)
                    base = plsc.load_gather(off32, [d])
                    pos = base + occ - 1
                    if p == 5:
                        plsc.store_scatter(sel_kf, [pos],
                                           plsc.bitcast(k, jnp.float32))
                    else:
                        plsc.store_scatter(dst_k, [pos], k)
                    plsc.store_scatter(dst_i, [pos], gi)
                    plsc.addupdate_scatter(off32, [d], occ, mask=lastm)
                    return 0
                lax.fori_loop(0, CAP // 16, permp, 0)

            pltpu.sync_copy(sel_kf, w_hbm.at[row])
            pltpu.sync_copy(selA_i, idx_hbm.at[row])
            return 0

        lax.fori_loop(0, 2, do_row, 0)

    return topk_kernel(scores_flat)


def kernel(hidden_states, weight):
    B, S, d = hidden_states.shape
    scores = _scores(hidden_states, weight)
    idx_flat, wgt_flat = _sc_topk(scores.reshape(B * NEXP, S))
    return (idx_flat.reshape(B, NEXP, CAP),
            wgt_flat.reshape(B, NEXP, CAP))
